# Initial kernel scaffold; baseline (speedup 1.0000x reference)
import jax
import jax.numpy as jnp
from jax.experimental import pallas as pl


def kernel(x, edge_index, edge_attr, batch, emb_table, W_rel, W_root, bias):
    raise NotImplementedError("write your pallas kernel here")


# speedup vs baseline: 5.1096x; 5.1096x over previous
import jax
import jax.numpy as jnp
from jax import lax
from jax.experimental import pallas as pl
from jax.experimental.pallas import tpu as pltpu
from jax.experimental.pallas import tpu_sc as plsc

N = 10000
E = 320000
D_IN = 128
D_OUT = 64
R = 16
VOCAB = 33
G = 256

VPAD = 64
ROOT_BASE = R * VOCAB * D_OUT
MWORDS = (R + 1) * VOCAB * D_OUT

NC = 2
NS = 16
NW = NC * NS
L = 16

C = 2000
CNT_PER_TILE = E // NS
MSG_PER_TILE = E // NW
NODE_GROUPS = N // L
NODE_GPW = (NODE_GROUPS + NW - 1) // NW
PW = G * D_OUT
PREG = PW // NS


def _prep_body(emb_ref, w_ref, out_ref):
    out_ref[0] = jnp.dot(emb_ref[...], w_ref[0],
                         preferred_element_type=jnp.float32)


def _prep(emb_padded, w_all):
    return pl.pallas_call(
        _prep_body,
        grid=(R + 1,),
        in_specs=[
            pl.BlockSpec((VPAD, D_IN), lambda j: (0, 0)),
            pl.BlockSpec((1, D_IN, D_OUT), lambda j: (j, 0, 0)),
        ],
        out_specs=pl.BlockSpec((1, VPAD, D_OUT), lambda j: (j, 0, 0)),
        out_shape=jax.ShapeDtypeStruct((R + 1, VPAD, D_OUT), jnp.float32),
    )(emb_padded, w_all)


def _sc_body(src_hbm, dst_hbm, rel_hbm, x_hbm, b_hbm, m_hbm,
             pool_out, ncnt_out,
             mtab_v, xtab_v, btab_v, pool_v, ncnt_v,
             srcb_v, relb_v, aux_v, seg_v, cnt_v, ones_v, zbuf_v,
             red_v, racc_v,
             cnt_s, pool_all_s, ncnt_all_s):
    cid = lax.axis_index("c")
    sid = lax.axis_index("s")
    wid = sid * NC + cid

    pltpu.sync_copy(m_hbm, mtab_v)
    pltpu.sync_copy(x_hbm, xtab_v)
    pltpu.sync_copy(b_hbm, btab_v)

    zeros16 = jnp.zeros((L,), jnp.float32)
    ones16 = jnp.ones((L,), jnp.float32)

    def _fill_z(i, _):
        zbuf_v[pl.ds(i * L, L)] = zeros16
        return 0
    lax.fori_loop(0, 10000 // L, _fill_z, 0)

    def _fill_pool(i, _):
        pool_v[pl.ds(i * L, L)] = zeros16
        return 0
    lax.fori_loop(0, PW // L, _fill_pool, 0)

    def _fill_nc(i, _):
        ncnt_v[pl.ds(i * L, L)] = zeros16
        return 0
    lax.fori_loop(0, G // L, _fill_nc, 0)

    def _fill_ones(i, _):
        ones_v[pl.ds(i * L, L)] = ones16
        return 0
    lax.fori_loop(0, C // L, _fill_ones, 0)

    cnt_slice = (N * R) // NS
    pltpu.sync_copy(zbuf_v, cnt_s.at[pl.ds(sid * cnt_slice, cnt_slice)])
    plsc.subcore_barrier()

    def _count_chunk(j, _):
        off = sid * CNT_PER_TILE + j * C
        pltpu.sync_copy(dst_hbm.at[pl.ds(off, C)], aux_v)
        pltpu.sync_copy(rel_hbm.at[pl.ds(off, C)], relb_v)

        def _seg(k, _):
            sl = pl.ds(k * L, L)
            seg_v[sl] = aux_v[sl] * R + relb_v[sl]
            return 0
        lax.fori_loop(0, C // L, _seg, 0)
        pltpu.sync_copy(ones_v, cnt_s.at[seg_v], add=True)
        return 0
    lax.fori_loop(0, CNT_PER_TILE // C, _count_chunk, 0)
    plsc.subcore_barrier()

    def _msg_chunk(j, _):
        off = wid * MSG_PER_TILE + j * C
        pltpu.sync_copy(src_hbm.at[pl.ds(off, C)], srcb_v)
        pltpu.sync_copy(dst_hbm.at[pl.ds(off, C)], aux_v)
        pltpu.sync_copy(rel_hbm.at[pl.ds(off, C)], relb_v)

        def _idx(k, _):
            sl = pl.ds(k * L, L)
            s16 = srcb_v[sl]
            d16 = aux_v[sl]
            r16 = relb_v[sl]
            seg_v[sl] = d16 * R + r16
            xs = plsc.load_gather(xtab_v, [s16])
            g16 = plsc.load_gather(btab_v, [d16])
            srcb_v[sl] = (r16 * VOCAB + xs) * D_OUT
            aux_v[sl] = g16 * D_OUT
            return 0
        lax.fori_loop(0, C // L, _idx, 0)

        pltpu.sync_copy(cnt_s.at[seg_v], cnt_v)

        def _accum(k, _):
            sl = pl.ds(k * L, L)
            nrm = 1.0 / jnp.maximum(cnt_v[sl], 1.0)
            b16 = srcb_v[sl]
            gb16 = aux_v[sl]
            for c in range(D_OUT):
                vals = plsc.load_gather(mtab_v, [b16 + c])
                plsc.addupdate_scatter(pool_v, [gb16 + c], vals * nrm)
            return 0
        lax.fori_loop(0, C // L, _accum, 0)
        return 0
    lax.fori_loop(0, MSG_PER_TILE // C, _msg_chunk, 0)

    ng = jnp.minimum(NODE_GROUPS - wid * NODE_GPW, NODE_GPW)

    def _node(k, _):
        off = (wid * NODE_GPW + k) * L
        sl = pl.ds(off, L)
        xs = xtab_v[sl]
        g16 = btab_v[sl]
        gb16 = g16 * D_OUT
        base = ROOT_BASE + xs * D_OUT
        plsc.addupdate_scatter(ncnt_v, [g16], ones16)
        for c in range(D_OUT):
            vals = plsc.load_gather(mtab_v, [base + c])
            plsc.addupdate_scatter(pool_v, [gb16 + c], vals)
        return 0
    lax.fori_loop(0, ng, _node, 0)

    pltpu.sync_copy(pool_v, pool_all_s.at[sid])
    pltpu.sync_copy(ncnt_v, ncnt_all_s.at[sid])
    plsc.subcore_barrier()

    def _racc0(i, _):
        racc_v[pl.ds(i * L, L)] = zeros16
        return 0
    lax.fori_loop(0, PREG // L, _racc0, 0)

    def _red_tile(t, _):
        pltpu.sync_copy(pool_all_s.at[t, pl.ds(sid * PREG, PREG)], red_v)

        def _vadd(i, _):
            sl = pl.ds(i * L, L)
            racc_v[sl] = racc_v[sl] + red_v[sl]
            return 0
        lax.fori_loop(0, PREG // L, _vadd, 0)
        return 0
    lax.fori_loop(0, NS, _red_tile, 0)
    pltpu.sync_copy(racc_v,
                    pool_out.at[pl.ds(cid * PW + sid * PREG, PREG)])

    @pl.when(sid == 0)
    def _():
        def _nred(t, _):
            pltpu.sync_copy(ncnt_all_s.at[t], red_v.at[pl.ds(0, G)])

            def _nadd(i, _):
                sl = pl.ds(i * L, L)
                racc_v[sl] = racc_v[sl] + red_v[sl]
                return 0
            lax.fori_loop(0, G // L, _nadd, 0)
            return 0

        def _nacc0(i, _):
            racc_v[pl.ds(i * L, L)] = zeros16
            return 0
        lax.fori_loop(0, G // L, _nacc0, 0)
        lax.fori_loop(0, NS, _nred, 0)
        pltpu.sync_copy(racc_v.at[pl.ds(0, G)],
                        ncnt_out.at[pl.ds(cid * G, G)])


def _sc_call(src, dst, rel, xflat, batch, mtab):
    mesh = plsc.VectorSubcoreMesh(core_axis_name="c", subcore_axis_name="s")
    kern = pl.kernel(
        _sc_body,
        out_type=(
            jax.ShapeDtypeStruct((NC * PW,), jnp.float32),
            jax.ShapeDtypeStruct((NC * G,), jnp.float32),
        ),
        mesh=mesh,
        compiler_params=pltpu.CompilerParams(needs_layout_passes=False),
        scratch_types=[
            pltpu.VMEM((MWORDS,), jnp.float32),
            pltpu.VMEM((N,), jnp.int32),
            pltpu.VMEM((N,), jnp.int32),
            pltpu.VMEM((PW,), jnp.float32),
            pltpu.VMEM((G,), jnp.float32),
            pltpu.VMEM((C,), jnp.int32),
            pltpu.VMEM((C,), jnp.int32),
            pltpu.VMEM((C,), jnp.int32),
            pltpu.VMEM((C,), jnp.int32),
            pltpu.VMEM((C,), jnp.float32),
            pltpu.VMEM((C,), jnp.float32),
            pltpu.VMEM((10000,), jnp.float32),
            pltpu.VMEM((PREG,), jnp.float32),
            pltpu.VMEM((PREG,), jnp.float32),
            pltpu.VMEM_SHARED((N * R,), jnp.float32),
            pltpu.VMEM_SHARED((NS, PW), jnp.float32),
            pltpu.VMEM_SHARED((NS, G), jnp.float32),
        ],
    )
    return kern(src, dst, rel, xflat, batch, mtab)


def _final_body(pp_ref, nc_ref, bias_ref, out_ref):
    n = nc_ref[0] + nc_ref[1]
    p = pp_ref[0] + pp_ref[1]
    out_ref[...] = ((p + n[:, None] * bias_ref[...][None, :])
                    / jnp.maximum(n, 1.0)[:, None])


def _final(pool_parts, ncnt_parts, bias):
    return pl.pallas_call(
        _final_body,
        out_shape=jax.ShapeDtypeStruct((G, D_OUT), jnp.float32),
    )(pool_parts, ncnt_parts, bias)


def kernel(x, edge_index, edge_attr, batch, emb_table, W_rel, W_root, bias):
    emb_padded = jnp.zeros((VPAD, D_IN), jnp.float32).at[:VOCAB].set(emb_table)
    w_all = jnp.concatenate([W_rel, W_root[None]], axis=0)
    m = _prep(emb_padded, w_all)
    mtab = m[:, :VOCAB, :].reshape(-1)

    src = edge_index[0].astype(jnp.int32)
    dst = edge_index[1].astype(jnp.int32)
    rel = edge_attr.reshape(-1).astype(jnp.int32)
    xflat = x.reshape(-1).astype(jnp.int32)
    batch_i = batch.astype(jnp.int32)

    pool_flat, ncnt_flat = _sc_call(src, dst, rel, xflat, batch_i, mtab)
    pool_parts = pool_flat.reshape(NC, G, D_OUT)
    ncnt_parts = ncnt_flat.reshape(NC, G)
    return _final(pool_parts, ncnt_parts, bias)


# speedup vs baseline: 45.3713x; 8.8797x over previous
import jax
import jax.numpy as jnp
from jax import lax
from jax.experimental import pallas as pl
from jax.experimental.pallas import tpu as pltpu
from jax.experimental.pallas import tpu_sc as plsc

N = 10000
E = 320000
D_IN = 128
D_OUT = 64
R = 16
VOCAB = 33
G = 256

VPAD = 64
WROWS = (R + 1) * VOCAB
ROOT_COL = R * VOCAB
TRASH_G = G

NC = 2
NS = 16
NW = NC * NS
L = 16

C = 2000
CNT_PER_TILE = E // NS
MSG_PER_TILE = E // NW
NPAD = NW * 320
WSLICE = 9024
WACC_WORDS = NS * WSLICE


def _prep_body(emb_ref, w_ref, out_ref):
    out_ref[0] = jnp.dot(emb_ref[...], w_ref[0],
                         preferred_element_type=jnp.float32)


def _prep(emb_padded, w_all):
    return pl.pallas_call(
        _prep_body,
        grid=(R + 1,),
        in_specs=[
            pl.BlockSpec((VPAD, D_IN), lambda j: (0, 0)),
            pl.BlockSpec((1, D_IN, D_OUT), lambda j: (j, 0, 0)),
        ],
        out_specs=pl.BlockSpec((1, VPAD, D_OUT), lambda j: (j, 0, 0)),
        out_shape=jax.ShapeDtypeStruct((R + 1, VPAD, D_OUT), jnp.float32),
    )(emb_padded, w_all)


def _sc_body(src_hbm, dst_hbm, rel_hbm, x_hbm, b_hbm,
             wacc_out,
             xtab_v, btab_v,
             srcb_v, relb_v, aux_v, seg_v, widx_v, cnt_v, ones_v, zbuf_v,
             nidx_v, nones_v, wslice_v,
             cnt_s, wacc_s):
    cid = lax.axis_index("c")
    sid = lax.axis_index("s")
    wid = sid * NC + cid

    pltpu.sync_copy(x_hbm, xtab_v)
    pltpu.sync_copy(b_hbm, btab_v)

    zeros16 = jnp.zeros((L,), jnp.float32)
    ones16 = jnp.ones((L,), jnp.float32)

    def _fill_z(i, _):
        zbuf_v[pl.ds(i * L, L)] = zeros16
        return 0
    lax.fori_loop(0, 10000 // L, _fill_z, 0)

    def _fill_ones(i, _):
        ones_v[pl.ds(i * L, L)] = ones16
        return 0
    lax.fori_loop(0, C // L, _fill_ones, 0)

    def _fill_nones(i, _):
        nones_v[pl.ds(i * L, L)] = ones16
        return 0
    lax.fori_loop(0, 320 // L, _fill_nones, 0)

    cnt_slice = (N * R) // NS
    pltpu.sync_copy(zbuf_v, cnt_s.at[pl.ds(sid * cnt_slice, cnt_slice)])
    pltpu.sync_copy(zbuf_v.at[pl.ds(0, WSLICE)],
                    wacc_s.at[pl.ds(sid * WSLICE, WSLICE)])
    plsc.subcore_barrier()

    def _count_chunk(j, _):
        off = sid * CNT_PER_TILE + j * C
        pltpu.sync_copy(dst_hbm.at[pl.ds(off, C)], aux_v)
        pltpu.sync_copy(rel_hbm.at[pl.ds(off, C)], relb_v)

        def _seg(k, _):
            sl = pl.ds(k * L, L)
            seg_v[sl] = aux_v[sl] * R + relb_v[sl]
            return 0
        lax.fori_loop(0, C // L, _seg, 0)
        pltpu.sync_copy(ones_v, cnt_s.at[seg_v], add=True)
        return 0
    lax.fori_loop(0, CNT_PER_TILE // C, _count_chunk, 0)
    plsc.subcore_barrier()

    def _msg_chunk(j, _):
        off = wid * MSG_PER_TILE + j * C
        pltpu.sync_copy(src_hbm.at[pl.ds(off, C)], srcb_v)
        pltpu.sync_copy(dst_hbm.at[pl.ds(off, C)], aux_v)
        pltpu.sync_copy(rel_hbm.at[pl.ds(off, C)], relb_v)

        def _idx(k, _):
            sl = pl.ds(k * L, L)
            s16 = srcb_v[sl]
            d16 = aux_v[sl]
            r16 = relb_v[sl]
            seg_v[sl] = d16 * R + r16
            xs = plsc.load_gather(xtab_v, [s16])
            g16 = plsc.load_gather(btab_v, [d16])
            widx_v[sl] = g16 * WROWS + r16 * VOCAB + xs
            return 0
        lax.fori_loop(0, C // L, _idx, 0)

        pltpu.sync_copy(cnt_s.at[seg_v], cnt_v)

        def _nrm(k, _):
            sl = pl.ds(k * L, L)
            cnt_v[sl] = 1.0 / jnp.maximum(cnt_v[sl], 1.0)
            return 0
        lax.fori_loop(0, C // L, _nrm, 0)
        pltpu.sync_copy(cnt_v, wacc_s.at[widx_v], add=True)
        return 0
    lax.fori_loop(0, MSG_PER_TILE // C, _msg_chunk, 0)

    def _node(k, _):
        sl = pl.ds(k * L, L)
        off = wid * 320 + k * L
        xs = xtab_v[pl.ds(off, L)]
        g16 = btab_v[pl.ds(off, L)]
        nidx_v[sl] = g16 * WROWS + ROOT_COL + xs
        return 0
    lax.fori_loop(0, 320 // L, _node, 0)
    pltpu.sync_copy(nones_v, wacc_s.at[nidx_v], add=True)
    plsc.subcore_barrier()

    pltpu.sync_copy(wacc_s.at[pl.ds(sid * WSLICE, WSLICE)], wslice_v)
    pltpu.sync_copy(wslice_v,
                    wacc_out.at[pl.ds((cid * NS + sid) * WSLICE, WSLICE)])


def _sc_call(src, dst, rel, xpad, bpad):
    mesh = plsc.VectorSubcoreMesh(core_axis_name="c", subcore_axis_name="s")
    kern = pl.kernel(
        _sc_body,
        out_type=jax.ShapeDtypeStruct((NC * WACC_WORDS,), jnp.float32),
        mesh=mesh,
        compiler_params=pltpu.CompilerParams(needs_layout_passes=False),
        scratch_types=[
            pltpu.VMEM((NPAD,), jnp.int32),
            pltpu.VMEM((NPAD,), jnp.int32),
            pltpu.VMEM((C,), jnp.int32),
            pltpu.VMEM((C,), jnp.int32),
            pltpu.VMEM((C,), jnp.int32),
            pltpu.VMEM((C,), jnp.int32),
            pltpu.VMEM((C,), jnp.int32),
            pltpu.VMEM((C,), jnp.float32),
            pltpu.VMEM((C,), jnp.float32),
            pltpu.VMEM((10000,), jnp.float32),
            pltpu.VMEM((320,), jnp.int32),
            pltpu.VMEM((320,), jnp.float32),
            pltpu.VMEM((WSLICE,), jnp.float32),
            pltpu.VMEM_SHARED((N * R,), jnp.float32),
            pltpu.VMEM_SHARED((WACC_WORDS,), jnp.float32),
        ],
    )
    return kern(src, dst, rel, xpad, bpad)


def _final_body(w_ref, m_ref, bias_ref, out_ref):
    w = w_ref[0] + w_ref[1]
    n = jnp.sum(w[:, ROOT_COL:WROWS], axis=1)
    pool = jnp.dot(w, m_ref[...], preferred_element_type=jnp.float32)
    out_ref[...] = ((pool + n[:, None] * bias_ref[...][None, :])
                    / jnp.maximum(n, 1.0)[:, None])


def _final(w_parts, mtab, bias):
    return pl.pallas_call(
        _final_body,
        out_shape=jax.ShapeDtypeStruct((G, D_OUT), jnp.float32),
    )(w_parts, mtab, bias)


def kernel(x, edge_index, edge_attr, batch, emb_table, W_rel, W_root, bias):
    emb_padded = jnp.zeros((VPAD, D_IN), jnp.float32).at[:VOCAB].set(emb_table)
    w_all = jnp.concatenate([W_rel, W_root[None]], axis=0)
    m = _prep(emb_padded, w_all)
    mtab = m[:, :VOCAB, :].reshape(WROWS, D_OUT)

    src = edge_index[0].astype(jnp.int32)
    dst = edge_index[1].astype(jnp.int32)
    rel = edge_attr.reshape(-1).astype(jnp.int32)
    xpad = jnp.concatenate(
        [x.reshape(-1).astype(jnp.int32),
         jnp.zeros((NPAD - N,), jnp.int32)])
    bpad = jnp.concatenate(
        [batch.astype(jnp.int32),
         jnp.full((NPAD - N,), TRASH_G, jnp.int32)])

    wacc = _sc_call(src, dst, rel, xpad, bpad)
    w_parts = wacc.reshape(NC, WACC_WORDS)[:, :G * WROWS].reshape(NC, G, WROWS)
    return _final(w_parts, mtab, bias)


# speedup vs baseline: 51.7784x; 1.1412x over previous
import jax
import jax.numpy as jnp
from jax import lax
from jax.experimental import pallas as pl
from jax.experimental.pallas import tpu as pltpu
from jax.experimental.pallas import tpu_sc as plsc

N = 10000
E = 320000
D_IN = 128
D_OUT = 64
R = 16
VOCAB = 33
G = 256

VPAD = 64
WROWS = (R + 1) * VOCAB
ROOT_COL = R * VOCAB
TRASH_G = G

NC = 2
NS = 16
NW = NC * NS
L = 16

C = 10000
CNT_PER_TILE = E // NS
MSG_PER_TILE = E // NW
NTPT = 320
NPAD = NW * NTPT
WSLICE = 9024
WACC_WORDS = NS * WSLICE


def _prep_body(emb_ref, w_ref, out_ref):
    out_ref[0] = jnp.dot(emb_ref[...], w_ref[0],
                         preferred_element_type=jnp.float32)


def _prep(emb_padded, w_all):
    return pl.pallas_call(
        _prep_body,
        grid=(R + 1,),
        in_specs=[
            pl.BlockSpec((VPAD, D_IN), lambda j: (0, 0)),
            pl.BlockSpec((1, D_IN, D_OUT), lambda j: (j, 0, 0)),
        ],
        out_specs=pl.BlockSpec((1, VPAD, D_OUT), lambda j: (j, 0, 0)),
        out_shape=jax.ShapeDtypeStruct((R + 1, VPAD, D_OUT), jnp.float32),
    )(emb_padded, w_all)


def _sc_body(src_hbm, dst_hbm, rel_hbm, x_hbm, b_hbm,
             wacc_out,
             xtab_v, btab_v,
             srcb_v, relb_v, aux_v, seg_v, widx_v, cnt_v, zbuf_v,
             nidx_v, nones_v, wslice_v,
             cnt_s, wacc_s):
    cid = lax.axis_index("c")
    sid = lax.axis_index("s")
    wid = sid * NC + cid

    with jax.named_scope("sc_setup"):
        pltpu.sync_copy(x_hbm, xtab_v.at[pl.ds(0, N)])
        pltpu.sync_copy(b_hbm, btab_v.at[pl.ds(0, N)])

        zeros16 = jnp.zeros((L,), jnp.float32)
        ones16 = jnp.ones((L,), jnp.float32)

        def _fill_z(i, _):
            zbuf_v[pl.ds(i * L, L)] = zeros16
            return 0
        lax.fori_loop(0, 10000 // L, _fill_z, 0)

        def _fill_nones(i, _):
            nones_v[pl.ds(i * L, L)] = ones16
            return 0
        lax.fori_loop(0, NTPT // L, _fill_nones, 0)

        cnt_slice = (N * R) // NS
        pltpu.sync_copy(zbuf_v, cnt_s.at[pl.ds(sid * cnt_slice, cnt_slice)])
        pltpu.sync_copy(zbuf_v.at[pl.ds(0, WSLICE)],
                        wacc_s.at[pl.ds(sid * WSLICE, WSLICE)])

        def _fill_ones(i, _):
            zbuf_v[pl.ds(i * L, L)] = ones16
            return 0
        lax.fori_loop(0, C // L, _fill_ones, 0)
    plsc.subcore_barrier()

    with jax.named_scope("sc_counts"):
        def _count_chunk(j, _):
            off = sid * CNT_PER_TILE + j * C
            pltpu.sync_copy(dst_hbm.at[pl.ds(off, C)], aux_v)
            pltpu.sync_copy(rel_hbm.at[pl.ds(off, C)], relb_v)

            def _seg(k, _):
                sl = pl.ds(k * L, L)
                seg_v[sl] = aux_v[sl] * R + relb_v[sl]
                return 0
            lax.fori_loop(0, C // L, _seg, 0)
            pltpu.sync_copy(zbuf_v, cnt_s.at[seg_v], add=True)
            return 0
        lax.fori_loop(0, CNT_PER_TILE // C, _count_chunk, 0)
    plsc.subcore_barrier()

    with jax.named_scope("sc_hist"):
        off = wid * MSG_PER_TILE
        pltpu.sync_copy(src_hbm.at[pl.ds(off, C)], srcb_v)
        pltpu.sync_copy(dst_hbm.at[pl.ds(off, C)], aux_v)
        pltpu.sync_copy(rel_hbm.at[pl.ds(off, C)], relb_v)

        def _idx(k, _):
            sl = pl.ds(k * L, L)
            s16 = srcb_v[sl]
            d16 = aux_v[sl]
            r16 = relb_v[sl]
            seg_v[sl] = d16 * R + r16
            xs = plsc.load_gather(xtab_v, [s16])
            g16 = plsc.load_gather(btab_v, [d16])
            widx_v[sl] = g16 * WROWS + r16 * VOCAB + xs
            return 0
        lax.fori_loop(0, C // L, _idx, 0)

        pltpu.sync_copy(cnt_s.at[seg_v], cnt_v)

        def _nrm(k, _):
            sl = pl.ds(k * L, L)
            cnt_v[sl] = 1.0 / jnp.maximum(cnt_v[sl], 1.0)
            return 0
        lax.fori_loop(0, C // L, _nrm, 0)
        pltpu.sync_copy(cnt_v, wacc_s.at[widx_v], add=True)

    with jax.named_scope("sc_nodes"):
        i16 = lax.iota(jnp.int32, L)
        trash = jnp.full((L,), TRASH_G * WROWS, jnp.int32)

        def _node(k, _):
            sl = pl.ds(k * L, L)
            noff = wid * NTPT + k * L
            xs = xtab_v[pl.ds(noff, L)]
            g16 = btab_v[pl.ds(noff, L)]
            widx = g16 * WROWS + ROOT_COL + xs
            nidx_v[sl] = jnp.where(noff + i16 < N, widx, trash)
            return 0
        lax.fori_loop(0, NTPT // L, _node, 0)
        pltpu.sync_copy(nones_v, wacc_s.at[nidx_v], add=True)
    plsc.subcore_barrier()

    with jax.named_scope("sc_out"):
        pltpu.sync_copy(wacc_s.at[pl.ds(sid * WSLICE, WSLICE)], wslice_v)
        pltpu.sync_copy(wslice_v,
                        wacc_out.at[pl.ds((cid * NS + sid) * WSLICE, WSLICE)])


def _sc_call(src, dst, rel, xpad, bpad):
    mesh = plsc.VectorSubcoreMesh(core_axis_name="c", subcore_axis_name="s")
    kern = pl.kernel(
        _sc_body,
        out_type=jax.ShapeDtypeStruct((NC * WACC_WORDS,), jnp.float32),
        mesh=mesh,
        compiler_params=pltpu.CompilerParams(needs_layout_passes=False),
        scratch_types=[
            pltpu.VMEM((NPAD,), jnp.int32),
            pltpu.VMEM((NPAD,), jnp.int32),
            pltpu.VMEM((C,), jnp.int32),
            pltpu.VMEM((C,), jnp.int32),
            pltpu.VMEM((C,), jnp.int32),
            pltpu.VMEM((C,), jnp.int32),
            pltpu.VMEM((C,), jnp.int32),
            pltpu.VMEM((C,), jnp.float32),
            pltpu.VMEM((10000,), jnp.float32),
            pltpu.VMEM((320,), jnp.int32),
            pltpu.VMEM((320,), jnp.float32),
            pltpu.VMEM((WSLICE,), jnp.float32),
            pltpu.VMEM_SHARED((N * R,), jnp.float32),
            pltpu.VMEM_SHARED((WACC_WORDS,), jnp.float32),
        ],
    )
    return kern(src, dst, rel, xpad, bpad)


def _final_body(w_ref, m_ref, bias_ref, out_ref):
    w = w_ref[0] + w_ref[1]
    n = jnp.sum(w[:, ROOT_COL:WROWS], axis=1)
    pool = jnp.dot(w, m_ref[...], preferred_element_type=jnp.float32)
    out_ref[...] = ((pool + n[:, None] * bias_ref[...][None, :])
                    / jnp.maximum(n, 1.0)[:, None])


def _final(w_parts, mtab, bias):
    return pl.pallas_call(
        _final_body,
        out_shape=jax.ShapeDtypeStruct((G, D_OUT), jnp.float32),
    )(w_parts, mtab, bias)


def kernel(x, edge_index, edge_attr, batch, emb_table, W_rel, W_root, bias):
    emb_padded = jnp.zeros((VPAD, D_IN), jnp.float32).at[:VOCAB].set(emb_table)
    w_all = jnp.concatenate([W_rel, W_root[None]], axis=0)
    m = _prep(emb_padded, w_all)
    mtab = m[:, :VOCAB, :].reshape(WROWS, D_OUT)

    src = edge_index[0].astype(jnp.int32)
    dst = edge_index[1].astype(jnp.int32)
    rel = edge_attr.reshape(-1).astype(jnp.int32)
    xflat = x.reshape(-1).astype(jnp.int32)
    batch_i = batch.astype(jnp.int32)

    wacc = _sc_call(src, dst, rel, xflat, batch_i)
    w_parts = wacc.reshape(NC, WACC_WORDS)[:, :G * WROWS].reshape(NC, G, WROWS)
    return _final(w_parts, mtab, bias)


# speedup vs baseline: 59.7508x; 1.1540x over previous
import jax
import jax.numpy as jnp
from jax import lax
from jax.experimental import pallas as pl
from jax.experimental.pallas import tpu as pltpu
from jax.experimental.pallas import tpu_sc as plsc

N = 10000
E = 320000
D_IN = 128
D_OUT = 64
R = 16
VOCAB = 33
G = 256

VPAD = 64
WSTRIDE = 564
WROWS = (R + 1) * VOCAB
ROOT_COL = R * VOCAB
TRASH_IDX = WROWS

NC = 2
NS = 16
NW = NC * NS
L = 16

C = 10000
CNT_PER_TILE = E // NS
MSG_PER_TILE = E // NW
NTPT = 320
NPAD = NW * NTPT
WACC_WORDS = G * WSTRIDE
WSLICE = WACC_WORDS // NS


def _sc_body(src_hbm, dst_hbm, rel_hbm, x_hbm, b_hbm,
             wacc_out,
             xtab_v, btab_v,
             srcb_v, relb_v, aux_v, seg_v, rel2_v, widx_v, cnt_v, zbuf_v,
             nidx_v, nones_v,
             sem_t, sem_z, sem_p, sem_c,
             cnt_s, wacc_s):
    cid = lax.axis_index("c")
    sid = lax.axis_index("s")
    wid = sid * NC + cid

    zeros16 = jnp.zeros((L,), jnp.float32)
    ones16 = jnp.ones((L,), jnp.float32)

    with jax.named_scope("sc_setup"):
        cp_x = pltpu.async_copy(x_hbm, xtab_v, sem_t)
        cp_b = pltpu.async_copy(b_hbm, btab_v, sem_t)

        def _fill_z(i, _):
            for u in range(8):
                zbuf_v[pl.ds((i * 8 + u) * L, L)] = zeros16
            return 0
        lax.fori_loop(0, WSLICE // (8 * L), _fill_z, 0)
        for u in range(WSLICE // (8 * L) * 8, WSLICE // L):
            zbuf_v[pl.ds(u * L, L)] = zeros16

        cnt_slice = (N * R) // NS
        cp_z1 = pltpu.async_copy(
            zbuf_v.at[pl.ds(0, 5000)],
            cnt_s.at[pl.ds(sid * cnt_slice, 5000)], sem_z)
        cp_z1b = pltpu.async_copy(
            zbuf_v.at[pl.ds(0, 5000)],
            cnt_s.at[pl.ds(sid * cnt_slice + 5000, 5000)], sem_z)
        cp_z2 = pltpu.async_copy(
            zbuf_v, wacc_s.at[pl.ds(sid * WSLICE, WSLICE)], sem_z)

        def _fill_nones(i, _):
            nones_v[pl.ds(i * L, L)] = ones16
            return 0
        lax.fori_loop(0, NTPT // L, _fill_nones, 0)

        cp_z1.wait()
        cp_z1b.wait()
        cp_z2.wait()

        def _fill_ones(i, _):
            for u in range(8):
                cnt_v[pl.ds((i * 8 + u) * L, L)] = ones16
            return 0
        lax.fori_loop(0, C // (8 * L), _fill_ones, 0)
        for u in range(C // (8 * L) * 8, C // L):
            cnt_v[pl.ds(u * L, L)] = ones16

        moff = wid * MSG_PER_TILE
        cp_p1 = pltpu.async_copy(src_hbm.at[pl.ds(moff, C)], srcb_v, sem_p)
        cp_p2 = pltpu.async_copy(dst_hbm.at[pl.ds(moff, C)], widx_v, sem_p)
        cp_p3 = pltpu.async_copy(rel_hbm.at[pl.ds(moff, C)], rel2_v, sem_p)

        cp_x.wait()
        cp_b.wait()
    plsc.subcore_barrier()

    with jax.named_scope("sc_counts"):
        off0 = sid * CNT_PER_TILE
        cp_l1 = pltpu.async_copy(dst_hbm.at[pl.ds(off0, C)], aux_v, sem_t)
        cp_l2 = pltpu.async_copy(rel_hbm.at[pl.ds(off0, C)], relb_v, sem_t)
        cp_l1.wait()
        cp_l2.wait()

        def _seg0(k, _):
            sl = pl.ds(k * L, L)
            seg_v[sl] = aux_v[sl] * R + relb_v[sl]
            return 0
        lax.fori_loop(0, C // L, _seg0, 0)
        cp_a0 = pltpu.async_copy(cnt_v, cnt_s.at[seg_v], sem_c, add=True)

        off1 = off0 + C
        cp_l3 = pltpu.async_copy(dst_hbm.at[pl.ds(off1, C)], aux_v, sem_t)
        cp_l4 = pltpu.async_copy(rel_hbm.at[pl.ds(off1, C)], relb_v, sem_t)
        cp_l3.wait()
        cp_l4.wait()
        cp_a0.wait()

        def _seg1(k, _):
            sl = pl.ds(k * L, L)
            seg_v[sl] = aux_v[sl] * R + relb_v[sl]
            return 0
        lax.fori_loop(0, C // L, _seg1, 0)
        cp_a1 = pltpu.async_copy(cnt_v, cnt_s.at[seg_v], sem_c, add=True)
        cp_a1.wait()
    plsc.subcore_barrier()

    with jax.named_scope("sc_hist"):
        cp_p1.wait()
        cp_p2.wait()
        cp_p3.wait()

        def _idx(k, _):
            sl = pl.ds(k * L, L)
            s16 = srcb_v[sl]
            d16 = widx_v[sl]
            r16 = rel2_v[sl]
            seg_v[sl] = d16 * R + r16
            xs = plsc.load_gather(xtab_v, [s16])
            g16 = plsc.load_gather(btab_v, [d16])
            aux_v[sl] = g16 * WSTRIDE + r16 * VOCAB + xs
            return 0
        lax.fori_loop(0, C // L, _idx, 0)

        pltpu.sync_copy(cnt_s.at[seg_v], cnt_v)

        def _nrm(k, _):
            sl = pl.ds(k * L, L)
            cnt_v[sl] = 1.0 / jnp.maximum(cnt_v[sl], 1.0)
            return 0
        lax.fori_loop(0, C // L, _nrm, 0)
        cp_w = pltpu.async_copy(cnt_v, wacc_s.at[aux_v], sem_c, add=True)

    with jax.named_scope("sc_nodes"):
        i16 = lax.iota(jnp.int32, L)
        trash = jnp.full((L,), TRASH_IDX, jnp.int32)

        def _node(k, _):
            sl = pl.ds(k * L, L)
            noff = wid * NTPT + k * L
            noff_c = jnp.minimum(noff, N - L)
            xs = xtab_v[pl.ds(noff_c, L)]
            g16 = btab_v[pl.ds(noff_c, L)]
            nw = g16 * WSTRIDE + ROOT_COL + xs
            nidx_v[sl] = jnp.where(noff + i16 < N, nw, trash)
            return 0
        lax.fori_loop(0, NTPT // L, _node, 0)
        cp_n = pltpu.async_copy(nones_v, wacc_s.at[nidx_v], sem_c, add=True)
        cp_w.wait()
        cp_n.wait()
    plsc.subcore_barrier()

    with jax.named_scope("sc_out"):
        pltpu.sync_copy(wacc_s.at[pl.ds(sid * WSLICE, WSLICE)], zbuf_v)
        pltpu.sync_copy(zbuf_v,
                        wacc_out.at[pl.ds((cid * NS + sid) * WSLICE, WSLICE)])


def _sc_call(src, dst, rel, xflat, batch):
    mesh = plsc.VectorSubcoreMesh(core_axis_name="c", subcore_axis_name="s")
    kern = pl.kernel(
        _sc_body,
        out_type=jax.ShapeDtypeStruct((NC * WACC_WORDS,), jnp.float32),
        mesh=mesh,
        compiler_params=pltpu.CompilerParams(needs_layout_passes=False),
        scratch_types=[
            pltpu.VMEM((N,), jnp.int32),
            pltpu.VMEM((N,), jnp.int32),
            pltpu.VMEM((C,), jnp.int32),
            pltpu.VMEM((C,), jnp.int32),
            pltpu.VMEM((C,), jnp.int32),
            pltpu.VMEM((C,), jnp.int32),
            pltpu.VMEM((C,), jnp.int32),
            pltpu.VMEM((C,), jnp.int32),
            pltpu.VMEM((C,), jnp.float32),
            pltpu.VMEM((WSLICE,), jnp.float32),
            pltpu.VMEM((NTPT,), jnp.int32),
            pltpu.VMEM((NTPT,), jnp.float32),
            pltpu.SemaphoreType.DMA,
            pltpu.SemaphoreType.DMA,
            pltpu.SemaphoreType.DMA,
            pltpu.SemaphoreType.DMA,
            pltpu.VMEM_SHARED((N * R,), jnp.float32),
            pltpu.VMEM_SHARED((WACC_WORDS,), jnp.float32),
        ],
    )
    return kern(src, dst, rel, xflat, batch)


def _prep_body(emb_ref, w_ref, out_ref):
    out_ref[0] = jnp.dot(emb_ref[...], w_ref[0],
                         preferred_element_type=jnp.float32)


def _prep(emb_padded, w_all):
    return pl.pallas_call(
        _prep_body,
        grid=(R + 1,),
        in_specs=[
            pl.BlockSpec((VPAD, D_IN), lambda j: (0, 0)),
            pl.BlockSpec((1, D_IN, D_OUT), lambda j: (j, 0, 0)),
        ],
        out_specs=pl.BlockSpec((1, VPAD, D_OUT), lambda j: (j, 0, 0)),
        out_shape=jax.ShapeDtypeStruct((R + 1, VPAD, D_OUT), jnp.float32),
    )(emb_padded, w_all)


def _final_body(w_ref, m_ref, bias_ref, out_ref):
    w = w_ref[0] + w_ref[1]
    n = jnp.sum(w[:, ROOT_COL:ROOT_COL + VOCAB], axis=1)
    pool = jnp.dot(w, m_ref[...], preferred_element_type=jnp.float32)
    out_ref[...] = ((pool + n[:, None] * bias_ref[...][None, :])
                    / jnp.maximum(n, 1.0)[:, None])


def _final(w_parts, m564, bias):
    return pl.pallas_call(
        _final_body,
        out_shape=jax.ShapeDtypeStruct((G, D_OUT), jnp.float32),
    )(w_parts, m564, bias)


def kernel(x, edge_index, edge_attr, batch, emb_table, W_rel, W_root, bias):
    emb_padded = jnp.zeros((VPAD, D_IN), jnp.float32).at[:VOCAB].set(emb_table)
    w_all = jnp.concatenate([W_rel, W_root[None]], axis=0)

    src = edge_index[0].astype(jnp.int32)
    dst = edge_index[1].astype(jnp.int32)
    rel = edge_attr.reshape(-1).astype(jnp.int32)
    xflat = x.reshape(-1).astype(jnp.int32)
    batch_i = batch.astype(jnp.int32)

    m = _prep(emb_padded, w_all)
    m564 = jnp.zeros((WSTRIDE, D_OUT), jnp.float32).at[:WROWS].set(
        m[:, :VOCAB, :].reshape(WROWS, D_OUT))

    wacc = _sc_call(src, dst, rel, xflat, batch_i)
    w_parts = wacc.reshape(NC, G, WSTRIDE)
    return _final(w_parts, m564, bias)
